# baseline (speedup 1.0000x reference)
import functools

import jax
import jax.numpy as jnp
from jax import lax
from jax.experimental import pallas as pl
from jax.experimental.pallas import tpu as pltpu
from jax.experimental.pallas import tpu_sc as plsc

EPS = 1e-8
QMAX = 128.0
G = 8192
GRID_R = 64
INV_W = 32.0


def _lut_build_body(centers_ref, thr_ref, tbl_ref):
    th = thr_ref[0]
    row = lax.broadcasted_iota(jnp.int32, (GRID_R, 128), 0)
    col = lax.broadcasted_iota(jnp.int32, (GRID_R, 128), 1)
    e_lo = (row * 128 + col).astype(jnp.float32) * (1.0 / INV_W) - QMAX
    e_hi = e_lo + (1.0 / INV_W)
    big = jnp.full((GRID_R, 128), 3.4e38, jnp.float32)
    zero = jnp.zeros((GRID_R, 128), jnp.float32)

    def body(k, carry):
        bd_lo, bv_lo, bd_hi, bv_hi = carry
        c = centers_ref[k]
        d_lo = jnp.abs(e_lo - c)
        d_hi = jnp.abs(e_hi - c)
        m_lo = d_lo < bd_lo
        m_hi = d_hi < bd_hi
        return (jnp.where(m_lo, d_lo, bd_lo), jnp.where(m_lo, c, bv_lo),
                jnp.where(m_hi, d_hi, bd_hi), jnp.where(m_hi, c, bv_hi))

    _, bv_lo, _, bv_hi = lax.fori_loop(0, 256, body, (big, zero, big, zero))
    x_per_t = (th + EPS) / QMAX
    out_scale = th / QMAX
    tbl_ref[pl.ds(0, GRID_R), :] = (bv_lo + bv_hi) * (0.5 * x_per_t)
    tbl_ref[pl.ds(GRID_R, GRID_R), :] = bv_lo * out_scale
    tbl_ref[pl.ds(2 * GRID_R, GRID_R), :] = bv_hi * out_scale
    a = (QMAX * INV_W) / (th + EPS)
    tbl_ref[pl.ds(3 * GRID_R, 8), :] = jnp.full((8, 128), a, jnp.float32)


_lut_build = pl.pallas_call(
    _lut_build_body,
    in_specs=[pl.BlockSpec(memory_space=pltpu.SMEM),
              pl.BlockSpec(memory_space=pltpu.SMEM)],
    out_specs=[pl.BlockSpec(memory_space=pltpu.VMEM)],
    out_shape=[jax.ShapeDtypeStruct((3 * GRID_R + 8, 128), jnp.float32)],
)


@functools.lru_cache(maxsize=None)
def _make_sc(n_rows, c, n_chunks):
    info = plsc.get_sparse_core_info()
    nc, ns = info.num_cores, info.num_subcores
    nw = nc * ns
    per_w = n_rows // nw
    ch = per_w // n_chunks
    assert per_w * nw == n_rows and ch * n_chunks == per_w and c % 16 == 0
    cols = []
    off = 0
    while off < c:
        cw = min(128, c - off)
        cols.append((off, cw))
        off += cw
    mesh = plsc.VectorSubcoreMesh(core_axis_name="c", subcore_axis_name="s")
    buf_types = []
    for _ in range(4):
        buf_types.extend(pltpu.VMEM((ch, cw), jnp.float32) for _, cw in cols)

    @functools.partial(
        pl.kernel,
        out_type=jax.ShapeDtypeStruct((n_rows, c), jnp.float32),
        mesh=mesh,
        scratch_types=[
            pltpu.VMEM(((3 * GRID_R + 8) * 128,), jnp.float32),
            *buf_types,
            pltpu.SemaphoreType.DMA,
            pltpu.SemaphoreType.DMA,
            pltpu.SemaphoreType.DMA,
            pltpu.SemaphoreType.DMA,
        ],
        compiler_params=pltpu.CompilerParams(needs_layout_passes=False),
    )
    def sc_fn(x_hbm, tbl_hbm, out_hbm, tbl_v, *rest):
        ncol = len(cols)
        bufs = rest[:4 * ncol]
        si0, si1, so0, so1 = rest[4 * ncol:]
        inb = (bufs[0:ncol], bufs[ncol:2 * ncol])
        outb = (bufs[2 * ncol:3 * ncol], bufs[3 * ncol:4 * ncol])
        sin, sout = (si0, si1), (so0, so1)
        wid = lax.axis_index("s") * nc + lax.axis_index("c")
        base = wid * per_w

        def in_copies(g, b):
            return [pltpu.make_async_copy(
                x_hbm.at[pl.ds(base + g * ch, ch), pl.ds(o, cw)],
                inb[b][k], sin[b]) for k, (o, cw) in enumerate(cols)]

        def out_copies(g, b):
            return [pltpu.make_async_copy(
                outb[b][k],
                out_hbm.at[pl.ds(base + g * ch, ch), pl.ds(o, cw)],
                sout[b]) for k, (o, cw) in enumerate(cols)]

        for cp in in_copies(0, 0):
            cp.start()
        pltpu.sync_copy(tbl_hbm, tbl_v)
        if n_chunks > 1:
            for cp in in_copies(1, 1):
                cp.start()
        av = tbl_v[pl.ds(3 * G, 16)]

        for g in range(n_chunks):
            b = g & 1
            for cp in in_copies(g, b):
                cp.wait()
            if g >= 2:
                for cp in out_copies(g - 2, b):
                    cp.wait()
            ibs, obs = inb[b], outb[b]

            @functools.partial(plsc.parallel_loop, 0, ch, unroll=4)
            def _(i, ibs=ibs, obs=obs):
                for k, (_, cw) in enumerate(cols):
                    for l in range(cw // 16):
                        xv = ibs[k][i, pl.ds(l * 16, 16)]
                        jf = xv * av + 4096.0
                        ji = jf.astype(jnp.int32)
                        ji = jnp.minimum(jnp.maximum(ji, 0), G - 1)
                        bpv = plsc.load_gather(tbl_v, [ji])
                        sel = jnp.where(xv > bpv, 2 * G, G)
                        vv = plsc.load_gather(tbl_v, [ji + sel])
                        obs[k][i, pl.ds(l * 16, 16)] = vv

            for cp in out_copies(g, b):
                cp.start()
            if g + 2 < n_chunks:
                for cp in in_copies(g + 2, b):
                    cp.start()
        if n_chunks >= 2:
            for cp in out_copies(n_chunks - 2, (n_chunks - 2) & 1):
                cp.wait()
        for cp in out_copies(n_chunks - 1, (n_chunks - 1) & 1):
            cp.wait()

    return sc_fn


def kernel(input_data, cluster_centers, threshold):
    [tbl2] = _lut_build(cluster_centers, threshold)
    tbl = tbl2.reshape(-1)
    b, h, w, c = input_data.shape
    xt = input_data.transpose(0, 1, 3, 2)
    n_rows = b * h * c
    x2 = xt.reshape(n_rows, w)
    out = _make_sc(n_rows, w, 28)(x2, tbl)
    return out.reshape(b, h, c, w).transpose(0, 1, 3, 2)


# speedup vs baseline: 1.0288x; 1.0288x over previous
import functools

import jax
import jax.numpy as jnp
from jax import lax
from jax.experimental import pallas as pl
from jax.experimental.pallas import tpu as pltpu
from jax.experimental.pallas import tpu_sc as plsc

EPS = 1e-8
QMAX = 128.0
G = 8192
GRID_R = 64
INV_W = 32.0


def _lut_build_body(centers_ref, thr_ref, tbl_ref):
    th = thr_ref[0]
    row = lax.broadcasted_iota(jnp.int32, (GRID_R, 128), 0)
    col = lax.broadcasted_iota(jnp.int32, (GRID_R, 128), 1)
    e_lo = (row * 128 + col).astype(jnp.float32) * (1.0 / INV_W) - QMAX
    e_hi = e_lo + (1.0 / INV_W)
    big = jnp.full((GRID_R, 128), 3.4e38, jnp.float32)
    zero = jnp.zeros((GRID_R, 128), jnp.float32)

    def body(k, carry):
        bd_lo, bv_lo, bd_hi, bv_hi = carry
        c = centers_ref[k]
        d_lo = jnp.abs(e_lo - c)
        d_hi = jnp.abs(e_hi - c)
        m_lo = d_lo < bd_lo
        m_hi = d_hi < bd_hi
        return (jnp.where(m_lo, d_lo, bd_lo), jnp.where(m_lo, c, bv_lo),
                jnp.where(m_hi, d_hi, bd_hi), jnp.where(m_hi, c, bv_hi))

    _, bv_lo, _, bv_hi = lax.fori_loop(0, 256, body, (big, zero, big, zero))
    x_per_t = (th + EPS) / QMAX
    out_scale = th / QMAX
    tbl_ref[pl.ds(0, GRID_R), :] = (bv_lo + bv_hi) * (0.5 * x_per_t)
    tbl_ref[pl.ds(GRID_R, GRID_R), :] = bv_lo * out_scale
    tbl_ref[pl.ds(2 * GRID_R, GRID_R), :] = bv_hi * out_scale
    a = (QMAX * INV_W) / (th + EPS)
    tbl_ref[pl.ds(3 * GRID_R, 8), :] = jnp.full((8, 128), a, jnp.float32)


_lut_build = pl.pallas_call(
    _lut_build_body,
    in_specs=[pl.BlockSpec(memory_space=pltpu.SMEM),
              pl.BlockSpec(memory_space=pltpu.SMEM)],
    out_specs=[pl.BlockSpec(memory_space=pltpu.VMEM)],
    out_shape=[jax.ShapeDtypeStruct((3 * GRID_R + 8, 128), jnp.float32)],
)


@functools.lru_cache(maxsize=None)
def _make_sc(n_rows, c, n_chunks):
    info = plsc.get_sparse_core_info()
    nc, ns = info.num_cores, info.num_subcores
    nw = nc * ns
    per_w = n_rows // nw
    ch = per_w // n_chunks
    assert per_w * nw == n_rows and ch * n_chunks == per_w and c % 16 == 0
    cols = []
    off = 0
    while off < c:
        cw = min(128, c - off)
        cols.append((off, cw))
        off += cw
    mesh = plsc.VectorSubcoreMesh(core_axis_name="c", subcore_axis_name="s")
    buf_types = []
    for _ in range(4):
        buf_types.extend(pltpu.VMEM((ch, cw), jnp.float32) for _, cw in cols)

    @functools.partial(
        pl.kernel,
        out_type=jax.ShapeDtypeStruct((n_rows, c), jnp.float32),
        mesh=mesh,
        scratch_types=[
            pltpu.VMEM(((3 * GRID_R + 8) * 128,), jnp.float32),
            *buf_types,
            pltpu.SemaphoreType.DMA,
            pltpu.SemaphoreType.DMA,
            pltpu.SemaphoreType.DMA,
            pltpu.SemaphoreType.DMA,
        ],
        compiler_params=pltpu.CompilerParams(needs_layout_passes=False),
    )
    def sc_fn(x_hbm, tbl_hbm, out_hbm, tbl_v, *rest):
        ncol = len(cols)
        bufs = rest[:4 * ncol]
        si0, si1, so0, so1 = rest[4 * ncol:]
        inb = (bufs[0:ncol], bufs[ncol:2 * ncol])
        outb = (bufs[2 * ncol:3 * ncol], bufs[3 * ncol:4 * ncol])
        sin, sout = (si0, si1), (so0, so1)
        wid = lax.axis_index("s") * nc + lax.axis_index("c")
        base = wid * per_w

        def in_copies(g, b):
            return [pltpu.make_async_copy(
                x_hbm.at[pl.ds(base + g * ch, ch), pl.ds(o, cw)],
                inb[b][k], sin[b]) for k, (o, cw) in enumerate(cols)]

        def out_copies(g, b):
            return [pltpu.make_async_copy(
                outb[b][k],
                out_hbm.at[pl.ds(base + g * ch, ch), pl.ds(o, cw)],
                sout[b]) for k, (o, cw) in enumerate(cols)]

        for cp in in_copies(0, 0):
            cp.start()
        pltpu.sync_copy(tbl_hbm, tbl_v)
        if n_chunks > 1:
            for cp in in_copies(1, 1):
                cp.start()
        av = tbl_v[pl.ds(3 * G, 16)]

        for g in range(n_chunks):
            b = g & 1
            for cp in in_copies(g, b):
                cp.wait()
            if g >= 2:
                for cp in out_copies(g - 2, b):
                    cp.wait()
            ibs, obs = inb[b], outb[b]

            @functools.partial(plsc.parallel_loop, 0, ch, unroll=2)
            def _(i, ibs=ibs, obs=obs):
                for k, (_, cw) in enumerate(cols):
                    for l in range(cw // 16):
                        xv = ibs[k][i, pl.ds(l * 16, 16)]
                        jf = xv * av + 4096.0
                        ji = jf.astype(jnp.int32)
                        ji = jnp.minimum(jnp.maximum(ji, 0), G - 1)
                        bpv = plsc.load_gather(tbl_v, [ji])
                        sel = jnp.where(xv > bpv, 2 * G, G)
                        vv = plsc.load_gather(tbl_v, [ji + sel])
                        obs[k][i, pl.ds(l * 16, 16)] = vv

            for cp in out_copies(g, b):
                cp.start()
            if g + 2 < n_chunks:
                for cp in in_copies(g + 2, b):
                    cp.start()
        if n_chunks >= 2:
            for cp in out_copies(n_chunks - 2, (n_chunks - 2) & 1):
                cp.wait()
        for cp in out_copies(n_chunks - 1, (n_chunks - 1) & 1):
            cp.wait()

    return sc_fn


def kernel(input_data, cluster_centers, threshold):
    [tbl2] = _lut_build(cluster_centers, threshold)
    tbl = tbl2.reshape(-1)
    b, h, w, c = input_data.shape
    xt = input_data.transpose(0, 1, 3, 2)
    n_rows = b * h * c
    x2 = xt.reshape(n_rows, w)
    out = _make_sc(n_rows, w, 14)(x2, tbl)
    return out.reshape(b, h, c, w).transpose(0, 1, 3, 2)
